# baseline (speedup 1.0000x reference)
import functools

import jax
import jax.numpy as jnp
from jax import lax
from jax.experimental import pallas as pl
from jax.experimental.pallas import tpu as pltpu
from jax.experimental.pallas import tpu_sc as plsc

_BATCH = 16384
_NC = 2
_NS = 16
_NW = _NC * _NS
_BPW = _BATCH // _NW

_mesh = plsc.VectorSubcoreMesh(core_axis_name="c", subcore_axis_name="s")


_NCH = 2
_CH = _BPW // _NCH


@functools.partial(
    pl.kernel,
    out_type=jax.ShapeDtypeStruct((_BATCH,), jnp.float32),
    mesh=_mesh,
    scratch_types=[
        pltpu.VMEM((_BPW,), jnp.int32),
        pltpu.VMEM((_BPW,), jnp.float32),
        [pltpu.SemaphoreType.DMA] * _NCH,
        [pltpu.SemaphoreType.DMA] * _NCH,
    ],
)
def _lookup(table_hbm, idx_hbm, out_hbm, idx_v, rows_v, gsems, osems):
    wid = lax.axis_index("s") * _NC + lax.axis_index("c")
    base = wid * _BPW
    pltpu.sync_copy(idx_hbm.at[pl.ds(base, _BPW)], idx_v)
    gathers = [
        pltpu.async_copy(
            table_hbm.at[idx_v.at[pl.ds(j * _CH, _CH)]],
            rows_v.at[pl.ds(j * _CH, _CH)],
            gsems[j],
        )
        for j in range(_NCH)
    ]
    outs = []
    for j in range(_NCH):
        gathers[j].wait()
        outs.append(
            pltpu.async_copy(
                rows_v.at[pl.ds(j * _CH, _CH)],
                out_hbm.at[pl.ds(base + j * _CH, _CH)],
                osems[j],
            )
        )
    for c in outs:
        c.wait()


def kernel(potential, states):
    return _lookup(potential, states.astype(jnp.int32))


# speedup vs baseline: 1.0035x; 1.0035x over previous
import functools

import jax
import jax.numpy as jnp
from jax import lax
from jax.experimental import pallas as pl
from jax.experimental.pallas import tpu as pltpu
from jax.experimental.pallas import tpu_sc as plsc

_BATCH = 16384
_NC = 2
_NS = 16
_NW = _NC * _NS
_BPW = _BATCH // _NW

_mesh = plsc.VectorSubcoreMesh(core_axis_name="c", subcore_axis_name="s")


@functools.partial(
    pl.kernel,
    out_type=jax.ShapeDtypeStruct((_BATCH,), jnp.float32),
    mesh=_mesh,
    scratch_types=[
        pltpu.VMEM((_BPW,), jnp.int32),
        pltpu.VMEM((_BPW,), jnp.float32),
        pltpu.SemaphoreType.DMA,
    ],
)
def _lookup(table_hbm, idx_hbm, out_hbm, idx_v, rows_v, sem):
    wid = lax.axis_index("c") * _NS + lax.axis_index("s")
    base = wid * _BPW
    pltpu.sync_copy(idx_hbm.at[pl.ds(base, _BPW)], idx_v)
    pltpu.async_copy(table_hbm.at[idx_v], rows_v, sem).wait()
    pltpu.sync_copy(rows_v, out_hbm.at[pl.ds(base, _BPW)])


def kernel(potential, states):
    return _lookup(potential, states.astype(jnp.int32))


# speedup vs baseline: 1.0238x; 1.0202x over previous
import dataclasses
import functools

import jax
import jax.numpy as jnp
from jax import lax
from jax.experimental import pallas as pl
from jax.experimental.pallas import tpu as pltpu
from jax.experimental.pallas import tpu_sc as plsc
from jax._src.pallas import mpmd
from jax._src.pallas import core as _pallas_core

_BATCH = 16384
_NC = 2
_NS = 16
_NW = _NC * _NS
_BPW = _BATCH // _NW
_PER_SC = _NS * _BPW

_scalar_mesh = plsc.ScalarSubcoreMesh(axis_name="c", num_cores=_NC)
_vector_mesh = plsc.VectorSubcoreMesh(core_axis_name="c", subcore_axis_name="s")


def _scs_fn(table_hbm, idx_hbm, out_hbm, idx_sp, ready, idx_v, rows_v, sem):
    del table_hbm, out_hbm, idx_v, rows_v, sem
    c = lax.axis_index("c")
    pltpu.sync_copy(idx_hbm.at[pl.ds(c * _PER_SC, _PER_SC)], idx_sp)
    for s in range(_NS):
        pltpu.semaphore_signal(ready, 1, device_id={"s": s})


def _tec_fn(table_hbm, idx_hbm, out_hbm, idx_sp, ready, idx_v, rows_v, sem):
    del idx_hbm
    c = lax.axis_index("c")
    s = lax.axis_index("s")
    pl.semaphore_wait(ready, 1)
    pltpu.sync_copy(idx_sp.at[pl.ds(s * _BPW, _BPW)], idx_v)
    pltpu.async_copy(table_hbm.at[idx_v], rows_v, sem).wait()
    base = (c * _NS + s) * _BPW
    pltpu.sync_copy(rows_v, out_hbm.at[pl.ds(base, _BPW)])


_lookup = mpmd.mpmd_map(
    [(_scalar_mesh, _scs_fn), (_vector_mesh, _tec_fn)],
    jax.ShapeDtypeStruct((_BATCH,), jnp.float32),
    scratch_types=[
        pltpu.VMEM_SHARED((_PER_SC,), jnp.int32),
        dataclasses.replace(
            pltpu.SemaphoreType.REGULAR(()),
            memory_space=_pallas_core.CoreMemorySpace(
                pltpu.MemorySpace.SEMAPHORE, _vector_mesh
            ),
        ),
        _pallas_core.CoreMemorySpace(pltpu.MemorySpace.VMEM, _vector_mesh)(
            (_BPW,), jnp.int32
        ),
        _pallas_core.CoreMemorySpace(pltpu.MemorySpace.VMEM, _vector_mesh)(
            (_BPW,), jnp.float32
        ),
        dataclasses.replace(
            pltpu.SemaphoreType.DMA(()),
            memory_space=_pallas_core.CoreMemorySpace(
                pltpu.MemorySpace.SEMAPHORE, _vector_mesh
            ),
        ),
    ],
)


def kernel(potential, states):
    return _lookup(potential, states.astype(jnp.int32))
